# baseline (speedup 1.0000x reference)
import functools

import jax
import jax.numpy as jnp
from jax import lax
from jax.experimental import pallas as pl
from jax.experimental.pallas import tpu as pltpu

G = 64
TOPG = 64
TOPK = 50


def _gmax_body(w_ref, d_ref, gmax_ref, *, gpb, pack, b):
    blk = d_ref[...]
    w = w_ref[...]
    s = lax.dot_general(blk, w, (((1,), (0,)), ((), ())),
                        preferred_element_type=jnp.float32)
    ppg = (G // pack)
    s3 = jnp.max(s.reshape(gpb, ppg, pack * b), axis=1)
    m = s3[:, :b]
    for j in range(1, pack):
        m = jnp.maximum(m, s3[:, j * b:(j + 1) * b])
    gmax_ref[...] = m[None]


def _topg_body(gmax_ref, gid_ref, scr_ref):
    scr_ref[...] = gmax_ref[...]
    c, b = scr_ref.shape

    def it(i, carry):
        v = scr_ref[...]
        a = jnp.argmax(v, axis=0).astype(jnp.int32)
        gid_ref[pl.ds(i, 1), :] = a[None, :]
        ii = lax.broadcasted_iota(jnp.int32, (c, b), 0)
        scr_ref[...] = jnp.where(ii == a[None, :], -jnp.inf, v)
        return carry

    lax.fori_loop(0, TOPG, it, 0)


def _rescore_body(gid_ref, qn_ref, d_ref, out_ref, scr_ref, sem):
    q = pl.program_id(0)
    for j in range(TOPG):
        g = gid_ref[q * TOPG + j]
        pltpu.make_async_copy(d_ref.at[g], scr_ref.at[pl.ds(j * G, G), :],
                              sem).start()
    for j in range(TOPG):
        pltpu.make_async_copy(d_ref.at[0], scr_ref.at[pl.ds(j * G, G), :],
                              sem).wait()
    qrow = qn_ref[pl.ds(q, 1), :]
    out_ref[...] = lax.dot_general(qrow, scr_ref[...],
                                   (((1,), (1,)), ((), ())),
                                   preferred_element_type=jnp.float32)[None]


def _select_body(cand_ref, ridx_ref, out_ref, scr_ref):
    scr_ref[...] = cand_ref[...]
    b, m = scr_ref.shape
    big = jnp.int32(1 << 30)

    def it(i, sel):
        s = scr_ref[...]
        ridx = ridx_ref[...]
        mx = jnp.max(s, axis=1, keepdims=True)
        tie = s == mx
        r = jnp.min(jnp.where(tie, ridx, big), axis=1, keepdims=True)
        scr_ref[...] = jnp.where(tie & (ridx == r), -jnp.inf, s)
        ii = lax.broadcasted_iota(jnp.int32, sel.shape, 1)
        return jnp.where(ii == i, r, sel)

    sel0 = jnp.zeros((b, TOPK), jnp.int32)
    out_ref[...] = lax.fori_loop(0, TOPK, it, sel0)


def kernel(query, data, n):
    if query.ndim == 1:
        query = query[None, :]
    b, d = query.shape
    nrows = data.shape[0]
    c = nrows // G
    pack = 4
    blk = 1600 if nrows % 1600 == 0 else nrows
    blk4 = blk // pack
    gpb = blk // G
    steps = nrows // blk

    norm = jnp.linalg.norm(query, ord=2, axis=1, keepdims=True)
    qn = query / jnp.clip(norm, 1e-12)

    w4 = jnp.kron(jnp.eye(pack, dtype=jnp.float32), qn.T)
    gmax = pl.pallas_call(
        functools.partial(_gmax_body, gpb=gpb, pack=pack, b=b),
        grid=(steps,),
        in_specs=[
            pl.BlockSpec((pack * d, pack * b), lambda i: (0, 0)),
            pl.BlockSpec((blk4, pack * d), lambda i: (i, 0)),
        ],
        out_specs=pl.BlockSpec((1, gpb, b), lambda i: (i, 0, 0)),
        out_shape=jax.ShapeDtypeStruct((steps, gpb, b), jnp.float32),
    )(w4, data.reshape(nrows // pack, pack * d))
    gmax = gmax.reshape(c, b)

    gid = pl.pallas_call(
        _topg_body,
        in_specs=[pl.BlockSpec((c, b), lambda: (0, 0))],
        out_specs=pl.BlockSpec((TOPG, b), lambda: (0, 0)),
        out_shape=jax.ShapeDtypeStruct((TOPG, b), jnp.int32),
        scratch_shapes=[pltpu.VMEM((c, b), jnp.float32)],
    )(gmax)

    gidt = gid.T
    cand = pl.pallas_call(
        _rescore_body,
        grid_spec=pltpu.PrefetchScalarGridSpec(
            num_scalar_prefetch=1,
            grid=(b,),
            in_specs=[
                pl.BlockSpec((b, d), lambda q, gref: (0, 0)),
                pl.BlockSpec(memory_space=pl.ANY),
            ],
            out_specs=pl.BlockSpec((1, 1, TOPG * G), lambda q, gref: (q, 0, 0)),
            scratch_shapes=[pltpu.VMEM((TOPG * G, d), jnp.float32),
                            pltpu.SemaphoreType.DMA],
        ),
        out_shape=jax.ShapeDtypeStruct((b, 1, TOPG * G), jnp.float32),
    )(gidt.reshape(-1), qn, data.reshape(c, G, d))
    cand = cand.reshape(b, TOPG * G)

    ridx = (gidt[:, :, None] * G
            + jnp.arange(G, dtype=jnp.int32)[None, None, :]).reshape(
                b, TOPG * G)

    sel = pl.pallas_call(
        _select_body,
        in_specs=[pl.BlockSpec((b, TOPG * G), lambda: (0, 0)),
                  pl.BlockSpec((b, TOPG * G), lambda: (0, 0))],
        out_specs=pl.BlockSpec((b, TOPK), lambda: (0, 0)),
        out_shape=jax.ShapeDtypeStruct((b, TOPK), jnp.int32),
        scratch_shapes=[pltpu.VMEM((b, TOPG * G), jnp.float32)],
    )(cand, ridx)

    idxs = sel.T
    if idxs.shape[1] == 1:
        return idxs[:, 0]
    return idxs


# speedup vs baseline: 1.6561x; 1.6561x over previous
import functools

import jax
import jax.numpy as jnp
from jax import lax
from jax.experimental import pallas as pl
from jax.experimental.pallas import tpu as pltpu

G = 64
TOPG = 64
TOPK = 50


def _gmax_body(qn_ref, d_ref, gmax_ref, *, gpb):
    blk = d_ref[...]
    qn = qn_ref[...]
    s = lax.dot_general(blk, qn, (((1,), (1,)), ((), ())),
                        preferred_element_type=jnp.float32)
    b = s.shape[1]
    gmax_ref[...] = jnp.max(s.reshape(gpb, G, b), axis=1)[None]


def _topg_body(gmax_ref, gid_ref, scr_ref):
    scr_ref[...] = gmax_ref[...]
    c, b = scr_ref.shape

    def it(i, aprev):
        v = scr_ref[...]
        ii = lax.broadcasted_iota(jnp.int32, (c, b), 0)
        vm = jnp.where(ii == aprev[None, :], -jnp.inf, v)
        scr_ref[...] = vm
        a = jnp.argmax(vm, axis=0).astype(jnp.int32)
        gid_ref[pl.ds(i, 1), :] = a[None, :]
        return a

    lax.fori_loop(0, TOPG, it, jnp.full((b,), -1, jnp.int32))


def _rescore_body(gid_ref, qn_ref, d_ref, out_ref, scr_ref, sem):
    q = pl.program_id(0)
    for j in range(TOPG):
        g = gid_ref[q * TOPG + j]
        pltpu.make_async_copy(d_ref.at[g], scr_ref.at[pl.ds(j * G, G), :],
                              sem).start()
    for j in range(TOPG):
        pltpu.make_async_copy(d_ref.at[0], scr_ref.at[pl.ds(j * G, G), :],
                              sem).wait()
    qrow = qn_ref[pl.ds(q, 1), :]
    out_ref[...] = lax.dot_general(qrow, scr_ref[...],
                                   (((1,), (1,)), ((), ())),
                                   preferred_element_type=jnp.float32)[None]


def _select_body(cand_ref, ridx_ref, out_ref, scr_ref):
    scr_ref[...] = cand_ref[...]
    b, m = scr_ref.shape
    big = jnp.int32(1 << 30)

    def it(i, carry):
        sel, mxp, rp = carry
        s = scr_ref[...]
        ridx = ridx_ref[...]
        s = jnp.where((s == mxp) & (ridx == rp), -jnp.inf, s)
        scr_ref[...] = s
        mx = jnp.max(s, axis=1, keepdims=True)
        r = jnp.min(jnp.where(s == mx, ridx, big), axis=1, keepdims=True)
        ii = lax.broadcasted_iota(jnp.int32, sel.shape, 1)
        return jnp.where(ii == i, r, sel), mx, r

    sel0 = (jnp.zeros((b, TOPK), jnp.int32),
            jnp.full((b, 1), jnp.inf, jnp.float32),
            jnp.full((b, 1), big, jnp.int32))
    out_ref[...] = lax.fori_loop(0, TOPK, it, sel0)[0]


def kernel(query, data, n):
    if query.ndim == 1:
        query = query[None, :]
    b, d = query.shape
    nrows = data.shape[0]
    c = nrows // G
    blk = 40000 if nrows % 40000 == 0 else nrows
    gpb = blk // G
    steps = nrows // blk

    norm = jnp.linalg.norm(query, ord=2, axis=1, keepdims=True)
    qn = query / jnp.clip(norm, 1e-12)

    gmax = pl.pallas_call(
        functools.partial(_gmax_body, gpb=gpb),
        grid=(steps,),
        in_specs=[
            pl.BlockSpec((b, d), lambda i: (0, 0)),
            pl.BlockSpec((blk, d), lambda i: (i, 0)),
        ],
        out_specs=pl.BlockSpec((1, gpb, b), lambda i: (i, 0, 0)),
        out_shape=jax.ShapeDtypeStruct((steps, gpb, b), jnp.float32),
    )(qn, data)
    gmax = gmax.reshape(c, b)

    gid = pl.pallas_call(
        _topg_body,
        in_specs=[pl.BlockSpec((c, b), lambda: (0, 0))],
        out_specs=pl.BlockSpec((TOPG, b), lambda: (0, 0)),
        out_shape=jax.ShapeDtypeStruct((TOPG, b), jnp.int32),
        scratch_shapes=[pltpu.VMEM((c, b), jnp.float32)],
    )(gmax)

    gidt = gid.T
    cand = pl.pallas_call(
        _rescore_body,
        grid_spec=pltpu.PrefetchScalarGridSpec(
            num_scalar_prefetch=1,
            grid=(b,),
            in_specs=[
                pl.BlockSpec((b, d), lambda q, gref: (0, 0)),
                pl.BlockSpec(memory_space=pl.ANY),
            ],
            out_specs=pl.BlockSpec((1, 1, TOPG * G), lambda q, gref: (q, 0, 0)),
            scratch_shapes=[pltpu.VMEM((TOPG * G, d), jnp.float32),
                            pltpu.SemaphoreType.DMA],
        ),
        out_shape=jax.ShapeDtypeStruct((b, 1, TOPG * G), jnp.float32),
    )(gidt.reshape(-1), qn, data.reshape(c, G, d))
    cand = cand.reshape(b, TOPG * G)

    ridx = (gidt[:, :, None] * G
            + jnp.arange(G, dtype=jnp.int32)[None, None, :]).reshape(
                b, TOPG * G)

    sel = pl.pallas_call(
        _select_body,
        in_specs=[pl.BlockSpec((b, TOPG * G), lambda: (0, 0)),
                  pl.BlockSpec((b, TOPG * G), lambda: (0, 0))],
        out_specs=pl.BlockSpec((b, TOPK), lambda: (0, 0)),
        out_shape=jax.ShapeDtypeStruct((b, TOPK), jnp.int32),
        scratch_shapes=[pltpu.VMEM((b, TOPG * G), jnp.float32)],
    )(cand, ridx)

    idxs = sel.T
    if idxs.shape[1] == 1:
        return idxs[:, 0]
    return idxs


# speedup vs baseline: 1.7247x; 1.0414x over previous
import functools

import jax
import jax.numpy as jnp
from jax import lax
from jax.experimental import pallas as pl
from jax.experimental.pallas import tpu as pltpu

G = 64
TOPG = 56
TOPK = 50


def _gmax_body(qn_ref, d_ref, gmax_ref, *, gpb):
    blk = d_ref[...]
    qn = qn_ref[...]
    s = lax.dot_general(blk, qn, (((1,), (1,)), ((), ())),
                        preferred_element_type=jnp.float32)
    b = s.shape[1]
    gmax_ref[...] = jnp.max(s.reshape(gpb, G, b), axis=1)[None]


def _topg_body(gmax_ref, gid_ref, scr_ref):
    scr_ref[...] = gmax_ref[...]
    c, b = scr_ref.shape

    def it(i, aprev):
        v = scr_ref[...]
        ii = lax.broadcasted_iota(jnp.int32, (c, b), 0)
        vm = jnp.where(ii == aprev[None, :], -jnp.inf, v)
        scr_ref[...] = vm
        a = jnp.argmax(vm, axis=0).astype(jnp.int32)
        gid_ref[pl.ds(i, 1), :] = a[None, :]
        return a

    lax.fori_loop(0, TOPG, it, jnp.full((b,), -1, jnp.int32))


def _rescore_body(gid_ref, qn_ref, d_ref, out_ref, scr_ref, sem):
    q = pl.program_id(0)
    for j in range(TOPG):
        g = gid_ref[q * TOPG + j]
        pltpu.make_async_copy(d_ref.at[g], scr_ref.at[pl.ds(j * G, G), :],
                              sem).start()
    for j in range(TOPG):
        pltpu.make_async_copy(d_ref.at[0], scr_ref.at[pl.ds(j * G, G), :],
                              sem).wait()
    qrow = qn_ref[pl.ds(q, 1), :]
    out_ref[...] = lax.dot_general(qrow, scr_ref[...],
                                   (((1,), (1,)), ((), ())),
                                   preferred_element_type=jnp.float32)[None]


def _select_body(cand_ref, ridx_ref, out_ref, scr_ref):
    scr_ref[...] = cand_ref[...]
    b, m = scr_ref.shape
    big = jnp.int32(1 << 30)

    def it(i, carry):
        sel, mxp, rp = carry
        s = scr_ref[...]
        ridx = ridx_ref[...]
        s = jnp.where((s == mxp) & (ridx == rp), -jnp.inf, s)
        scr_ref[...] = s
        mx = jnp.max(s, axis=1, keepdims=True)
        r = jnp.min(jnp.where(s == mx, ridx, big), axis=1, keepdims=True)
        ii = lax.broadcasted_iota(jnp.int32, sel.shape, 1)
        return jnp.where(ii == i, r, sel), mx, r

    sel0 = (jnp.zeros((b, TOPK), jnp.int32),
            jnp.full((b, 1), jnp.inf, jnp.float32),
            jnp.full((b, 1), big, jnp.int32))
    out_ref[...] = lax.fori_loop(0, TOPK, it, sel0)[0]


def kernel(query, data, n):
    if query.ndim == 1:
        query = query[None, :]
    b, d = query.shape
    nrows = data.shape[0]
    c = nrows // G
    blk = 40000 if nrows % 40000 == 0 else nrows
    gpb = blk // G
    steps = nrows // blk

    norm = jnp.linalg.norm(query, ord=2, axis=1, keepdims=True)
    qn = query / jnp.clip(norm, 1e-12)

    gmax = pl.pallas_call(
        functools.partial(_gmax_body, gpb=gpb),
        grid=(steps,),
        in_specs=[
            pl.BlockSpec((b, d), lambda i: (0, 0)),
            pl.BlockSpec((blk, d), lambda i: (i, 0)),
        ],
        out_specs=pl.BlockSpec((1, gpb, b), lambda i: (i, 0, 0)),
        out_shape=jax.ShapeDtypeStruct((steps, gpb, b), jnp.float32),
    )(qn, data)
    gmax = gmax.reshape(c, b)

    gid = pl.pallas_call(
        _topg_body,
        in_specs=[pl.BlockSpec((c, b), lambda: (0, 0))],
        out_specs=pl.BlockSpec((TOPG, b), lambda: (0, 0)),
        out_shape=jax.ShapeDtypeStruct((TOPG, b), jnp.int32),
        scratch_shapes=[pltpu.VMEM((c, b), jnp.float32)],
    )(gmax)

    gidt = gid.T
    cand = pl.pallas_call(
        _rescore_body,
        grid_spec=pltpu.PrefetchScalarGridSpec(
            num_scalar_prefetch=1,
            grid=(b,),
            in_specs=[
                pl.BlockSpec((b, d), lambda q, gref: (0, 0)),
                pl.BlockSpec(memory_space=pl.ANY),
            ],
            out_specs=pl.BlockSpec((1, 1, TOPG * G), lambda q, gref: (q, 0, 0)),
            scratch_shapes=[pltpu.VMEM((TOPG * G, d), jnp.float32),
                            pltpu.SemaphoreType.DMA],
        ),
        out_shape=jax.ShapeDtypeStruct((b, 1, TOPG * G), jnp.float32),
    )(gidt.reshape(-1), qn, data.reshape(c, G, d))
    cand = cand.reshape(b, TOPG * G)

    ridx = (gidt[:, :, None] * G
            + jnp.arange(G, dtype=jnp.int32)[None, None, :]).reshape(
                b, TOPG * G)

    sel = pl.pallas_call(
        _select_body,
        in_specs=[pl.BlockSpec((b, TOPG * G), lambda: (0, 0)),
                  pl.BlockSpec((b, TOPG * G), lambda: (0, 0))],
        out_specs=pl.BlockSpec((b, TOPK), lambda: (0, 0)),
        out_shape=jax.ShapeDtypeStruct((b, TOPK), jnp.int32),
        scratch_shapes=[pltpu.VMEM((b, TOPG * G), jnp.float32)],
    )(cand, ridx)

    idxs = sel.T
    if idxs.shape[1] == 1:
        return idxs[:, 0]
    return idxs
